# baseline (speedup 1.0000x reference)
import functools

import jax
import jax.numpy as jnp
from jax import lax
from jax.experimental import pallas as pl
from jax.experimental.pallas import tpu as pltpu
from jax.experimental.pallas import tpu_sc as plsc

NC, NS = 2, 16
NW = NC * NS
CHUNK = 64
D = 128
DA = 144
BLK = 640


SLAB = 8
STEP = 2 * SLAB


def _sc_pass_body(K, stripe, tbl, ei, zeros, out, slab_a, slab_b,
                  v0, v1, v2, v3, acc,
                  sg0, sg1, sg2, sg3, ss0, ss1, ss2, ss3, si):
    cid = lax.axis_index("c")
    sid = lax.axis_index("s")
    wid = cid * NS + sid
    n_acc = acc.shape[0]
    vals = [v0, v1, v2, v3]
    sg = [sg0, sg1, sg2, sg3]
    ss = [ss0, ss1, ss2, ss3]
    iters = K // STEP

    def slab(p):
        return slab_a if p == 0 else slab_b

    def gather_issue(p, r, s):
        pltpu.async_copy(tbl.at[slab(p).at[r, 0]], vals[s], sg[s])

    def gather_wait(p, r, s):
        pltpu.make_async_copy(tbl.at[slab(p).at[r, 0]], vals[s], sg[s]).wait()

    def scat_issue(p, r, s):
        pltpu.async_copy(vals[s], acc.at[slab(p).at[r, 1]], ss[s], add=True)

    def scat_wait(s):
        pltpu.make_async_copy(vals[s], acc.at[slab_a.at[0, 1]], ss[s]).wait()

    def slab_issue(base, buf):
        pltpu.async_copy(ei.at[wid].at[pl.ds(base, SLAB)], buf, si)

    def slab_wait(buf):
        pltpu.make_async_copy(ei.at[wid].at[pl.ds(0, SLAB)], buf, si).wait()

    pltpu.sync_copy(zeros.at[pl.ds(sid * stripe, stripe)],
                    acc.at[pl.ds(sid * stripe, stripe)])
    pltpu.sync_copy(ei.at[wid].at[pl.ds(0, SLAB)], slab_a)
    gather_issue(0, 0, 0)
    gather_issue(0, 1, 1)
    plsc.subcore_barrier()

    def step(i, carry):
        g16 = STEP * i
        for t in range(STEP):
            if t >= 2:
                scat_wait((t - 2) % 4)
            else:
                @pl.when(i > 0)
                def _():
                    scat_wait((t - 2) % 4)
            if t == 1:
                slab_issue(g16 + SLAB, slab_b)
            if t == 6:
                slab_wait(slab_b)
            if t == 9:
                @pl.when(i < iters - 1)
                def _():
                    slab_issue(g16 + 2 * SLAB, slab_a)
            if t < STEP - 2:
                gather_issue(((t + 2) // SLAB) % 2, (t + 2) % SLAB, (t + 2) % 4)
            else:
                @pl.when(i < iters - 1)
                def _():
                    if t == STEP - 2:
                        slab_wait(slab_a)
                    gather_issue(0, t + 2 - STEP, (t + 2) % 4)
            gather_wait(t // SLAB, t % SLAB, t % 4)
            scat_issue(t // SLAB, t % SLAB, t % 4)
        return carry

    lax.fori_loop(0, iters, step, 0)
    scat_wait((K - 2) % 4)
    scat_wait((K - 1) % 4)
    plsc.subcore_barrier()
    pltpu.sync_copy(acc.at[pl.ds(sid * stripe, stripe)],
                    out.at[pl.ds(cid * n_acc + sid * stripe, stripe)])


def _make_sc_pass(n_acc, K):
    stripe = n_acc // NS
    mesh = plsc.VectorSubcoreMesh(core_axis_name="c", subcore_axis_name="s",
                                  num_cores=NC, num_subcores=NS)
    dma = pltpu.SemaphoreType.DMA
    return pl.kernel(
        functools.partial(_sc_pass_body, K, stripe),
        out_type=jax.ShapeDtypeStruct((NC * n_acc, DA), jnp.float32),
        mesh=mesh,
        scratch_types=[
            pltpu.VMEM((SLAB, 2, CHUNK), jnp.int32),
            pltpu.VMEM((SLAB, 2, CHUNK), jnp.int32),
            pltpu.VMEM((CHUNK, DA), jnp.float32),
            pltpu.VMEM((CHUNK, DA), jnp.float32),
            pltpu.VMEM((CHUNK, DA), jnp.float32),
            pltpu.VMEM((CHUNK, DA), jnp.float32),
            pltpu.VMEM_SHARED((n_acc, DA), jnp.float32),
            dma, dma, dma, dma,
            dma, dma, dma, dma,
            dma,
        ],
        compiler_params=pltpu.CompilerParams(use_tc_tiling_on_sc=False),
    )


def _ones_col(rows):
    return (lax.broadcasted_iota(jnp.int32, (rows, DA - D), 1) == 0).astype(jnp.float32)


def _k1_body(x_ref, w_ref, o_ref):
    m = jnp.dot(x_ref[...], w_ref[...], preferred_element_type=jnp.float32)
    o_ref[...] = jnp.concatenate([m, _ones_col(m.shape[0])], axis=1)


def _k2_body(acc_ref, w_ref, o_ref):
    p = acc_ref[0] + acc_ref[1]
    ef = p[:, :D] / jnp.maximum(p[:, D:D + 1], 1.0)
    ep = jnp.dot(ef, w_ref[...], preferred_element_type=jnp.float32)
    o_ref[...] = jnp.concatenate([ep, _ones_col(ep.shape[0])], axis=1)


def _k3_body(xa_ref, acc_ref, wu_ref, b_ref, o_ref):
    p = acc_ref[0] + acc_ref[1]
    nagg = p[:, :D] / jnp.maximum(p[:, D:D + 1], 1.0)
    h = (jnp.dot(xa_ref[:, :D], wu_ref[:D], preferred_element_type=jnp.float32)
         + jnp.dot(nagg, wu_ref[D:], preferred_element_type=jnp.float32)
         + b_ref[...])
    o_ref[...] = jnp.maximum(h, 0.0)


def kernel(x, edge_index, W_v, W_e, W_upd, b_upd):
    n = x.shape[0]
    e = edge_index.shape[1]
    n_pad = ((n + 1 + BLK - 1) // BLK) * BLK
    n_acc = ((n + 1 + NS - 1) // NS) * NS
    K = -(-e // (NW * CHUNK))
    K += (-K) % STEP
    e_pad = NW * K * CHUNK

    row = edge_index[0]
    col = edge_index[1]
    fill = jnp.full((e_pad - e,), n, jnp.int32)
    row_p = jnp.concatenate([row, fill]).reshape(NW, K, CHUNK)
    col_p = jnp.concatenate([col, fill]).reshape(NW, K, CHUNK)
    ei_a = jnp.stack([row_p, col_p], axis=2)
    ei_b = jnp.stack([col_p, row_p], axis=2)

    x_pad = jnp.zeros((n_pad, D), jnp.float32).at[:n].set(x)
    zeros = jnp.zeros((n_acc, DA), jnp.float32)

    grid = n_pad // BLK
    full = lambda shape: pl.BlockSpec(shape, lambda i: (0,) * len(shape))

    x_self_aug = pl.pallas_call(
        _k1_body,
        grid=(grid,),
        in_specs=[pl.BlockSpec((BLK, D), lambda i: (i, 0)), full((D, D))],
        out_specs=pl.BlockSpec((BLK, DA), lambda i: (i, 0)),
        out_shape=jax.ShapeDtypeStruct((n_pad, DA), jnp.float32),
    )(x_pad, W_v)

    sc_pass = _make_sc_pass(n_acc, K)
    acc_a = sc_pass(x_self_aug, ei_a, zeros).reshape(NC, n_acc, DA)

    e_proj_aug = pl.pallas_call(
        _k2_body,
        grid=(grid,),
        in_specs=[pl.BlockSpec((NC, BLK, DA), lambda i: (0, i, 0)), full((D, D))],
        out_specs=pl.BlockSpec((BLK, DA), lambda i: (i, 0)),
        out_shape=jax.ShapeDtypeStruct((n_pad, DA), jnp.float32),
    )(acc_a, W_e)

    acc_b = sc_pass(e_proj_aug, ei_b, zeros).reshape(NC, n_acc, DA)

    out = pl.pallas_call(
        _k3_body,
        grid=(grid,),
        in_specs=[
            pl.BlockSpec((BLK, DA), lambda i: (i, 0)),
            pl.BlockSpec((NC, BLK, DA), lambda i: (0, i, 0)),
            full((2 * D, D)),
            full((1, D)),
        ],
        out_specs=pl.BlockSpec((BLK, D), lambda i: (i, 0)),
        out_shape=jax.ShapeDtypeStruct((n_pad, D), jnp.float32),
    )(x_self_aug, acc_b, W_upd, b_upd.reshape(1, D))

    return out[:n]


# speedup vs baseline: 1.6468x; 1.6468x over previous
import functools

import jax
import jax.numpy as jnp
from jax import lax
from jax.experimental import pallas as pl
from jax.experimental.pallas import tpu as pltpu
from jax.experimental.pallas import tpu_sc as plsc

NC, NS = 2, 16
NW = NC * NS
CHUNK = 64
D = 128
BLK = 1024
HR = 80


def _sc_pass_body(K, stripe, tbl, gidx, sidx, zeros, out, cnt_out,
                  gv, sv, vals0, vals1, hist, acc, sem):
    cid = lax.axis_index("c")
    sid = lax.axis_index("s")
    wid = cid * NS + sid
    n_acc = acc.shape[0]
    pltpu.sync_copy(zeros.at[pl.ds(sid * stripe, stripe)],
                    acc.at[pl.ds(sid * stripe, stripe)])

    def zro(i, carry):
        hist[pl.ds(i * 16, 16)] = jnp.zeros((16,), jnp.float32)
        return carry

    lax.fori_loop(0, n_acc // 16, zro, 0)
    pltpu.sync_copy(gidx.at[wid], gv)
    pltpu.sync_copy(sidx.at[wid], sv)
    plsc.subcore_barrier()

    ones = jnp.ones((16,), jnp.float32)

    def count(j):
        for k in range(CHUNK // 16):
            idx_v = sv[j, pl.ds(16 * k, 16)]
            plsc.addupdate_scatter(hist, [idx_v], ones)

    pltpu.async_copy(tbl.at[gv.at[0]], vals0, sem)

    def step(i, carry):
        j = 2 * i
        pltpu.make_async_copy(tbl.at[gv.at[j]], vals0, sem).wait()
        pltpu.async_copy(tbl.at[gv.at[j + 1]], vals1, sem)
        pltpu.sync_copy(vals0, acc.at[sv.at[j]], add=True)
        count(j)
        pltpu.make_async_copy(tbl.at[gv.at[j + 1]], vals1, sem).wait()

        @pl.when(j + 2 < K)
        def _():
            pltpu.async_copy(tbl.at[gv.at[j + 2]], vals0, sem)

        pltpu.sync_copy(vals1, acc.at[sv.at[j + 1]], add=True)
        count(j + 1)
        return carry

    lax.fori_loop(0, K // 2, step, 0)
    plsc.subcore_barrier()
    pltpu.sync_copy(acc.at[pl.ds(sid * stripe, stripe)],
                    out.at[pl.ds(cid * n_acc + sid * stripe, stripe)])
    pltpu.sync_copy(hist, cnt_out.at[wid])


def _make_sc_pass(n_acc, K):
    stripe = n_acc // NS
    mesh = plsc.VectorSubcoreMesh(core_axis_name="c", subcore_axis_name="s",
                                  num_cores=NC, num_subcores=NS)
    return pl.kernel(
        functools.partial(_sc_pass_body, K, stripe),
        out_type=(jax.ShapeDtypeStruct((NC * n_acc, D), jnp.float32),
                  jax.ShapeDtypeStruct((NW, n_acc), jnp.float32)),
        mesh=mesh,
        scratch_types=[
            pltpu.VMEM((K, CHUNK), jnp.int32),
            pltpu.VMEM((K, CHUNK), jnp.int32),
            pltpu.VMEM((CHUNK, D), jnp.float32),
            pltpu.VMEM((CHUNK, D), jnp.float32),
            pltpu.VMEM((n_acc,), jnp.float32),
            pltpu.VMEM_SHARED((n_acc, D), jnp.float32),
            pltpu.SemaphoreType.DMA,
        ],
        compiler_params=pltpu.CompilerParams(use_tc_tiling_on_sc=False,
                                             needs_layout_passes=False),
    )


def _k1_body(x_ref, w_ref, o_ref):
    o_ref[...] = jnp.dot(x_ref[...], w_ref[...],
                         preferred_element_type=jnp.float32)


def _mean(acc_blk, cnt_blk):
    sums = acc_blk[0] + acc_blk[1]
    cnt = jnp.sum(cnt_blk[...], axis=0, keepdims=True)
    cnt = jnp.transpose(jnp.maximum(cnt, 1.0))
    return sums / cnt


def _k2_body(acc_ref, cnt_ref, w_ref, o_ref):
    ef = _mean(acc_ref, cnt_ref)
    o_ref[...] = jnp.dot(ef, w_ref[...], preferred_element_type=jnp.float32)


def _k3_body(xs_ref, acc_ref, cnt_ref, wu_ref, b_ref, o_ref):
    nagg = _mean(acc_ref, cnt_ref)
    h = (jnp.dot(xs_ref[...], wu_ref[:D], preferred_element_type=jnp.float32)
         + jnp.dot(nagg, wu_ref[D:], preferred_element_type=jnp.float32)
         + b_ref[...])
    o_ref[...] = jnp.maximum(h, 0.0)


def kernel(x, edge_index, W_v, W_e, W_upd, b_upd):
    n = x.shape[0]
    e = edge_index.shape[1]
    n_pad = ((n + 1 + BLK - 1) // BLK) * BLK
    K = -(-e // (NW * CHUNK))
    K += K % 2
    e_pad = NW * K * CHUNK

    row = edge_index[0]
    col = edge_index[1]
    fill = jnp.full((e_pad - e,), n, jnp.int32)
    row_p = jnp.concatenate([row, fill]).reshape(NW, K, CHUNK)
    col_p = jnp.concatenate([col, fill]).reshape(NW, K, CHUNK)

    x_pad = jnp.zeros((n_pad, D), jnp.float32).at[:n].set(x)
    zeros = jnp.zeros((n_pad, D), jnp.float32)

    grid = n_pad // BLK
    full = lambda shape: pl.BlockSpec(shape, lambda i: (0,) * len(shape))
    rows = pl.BlockSpec((BLK, D), lambda i: (i, 0))
    accs = pl.BlockSpec((NC, BLK, D), lambda i: (0, i, 0))
    cnts = pl.BlockSpec((NW, BLK), lambda i: (0, i))

    x_self = pl.pallas_call(
        _k1_body,
        grid=(grid,),
        in_specs=[rows, full((D, D))],
        out_specs=rows,
        out_shape=jax.ShapeDtypeStruct((n_pad, D), jnp.float32),
    )(x_pad, W_v)

    sc_pass = _make_sc_pass(n_pad, K)
    acc_a, cnt_a = sc_pass(x_self, row_p, col_p, zeros)
    acc_a = acc_a.reshape(NC, n_pad, D)

    e_proj = pl.pallas_call(
        _k2_body,
        grid=(grid,),
        in_specs=[accs, cnts, full((D, D))],
        out_specs=rows,
        out_shape=jax.ShapeDtypeStruct((n_pad, D), jnp.float32),
    )(acc_a, cnt_a, W_e)

    acc_b, cnt_b = sc_pass(e_proj, col_p, row_p, zeros)
    acc_b = acc_b.reshape(NC, n_pad, D)

    out = pl.pallas_call(
        _k3_body,
        grid=(grid,),
        in_specs=[rows, accs, cnts, full((2 * D, D)), full((1, D))],
        out_specs=rows,
        out_shape=jax.ShapeDtypeStruct((n_pad, D), jnp.float32),
    )(x_self, acc_b, cnt_b, W_upd, b_upd.reshape(1, D))

    return out[:n]
